# baseline (speedup 1.0000x reference)
import jax
import jax.numpy as jnp
from jax.experimental import pallas as pl
from jax.experimental.pallas import tpu as pltpu

N = 20000
NP = 20480
G = 128
TOP_K_N = 200
R, C = 8, 2560
CHUNK = 5120
CS = 208
BIG = 2 ** 30
FG = 0.8
BG = 0.1


def _matcher_body(a_ref, gt_ref, labels_ref, boxes_ref, iou_ref, onehot_ref):
    gx1 = gt_ref[:, 0:1]
    gy1 = gt_ref[:, 1:2]
    gx2 = gt_ref[:, 2:3]
    gy2 = gt_ref[:, 3:4]
    area_g = (gx2 - gx1) * (gy2 - gy1)

    aT = jnp.concatenate(
        [jnp.transpose(a_ref[:]), jnp.zeros((4, NP - N), jnp.float32)],
        axis=1)

    mv_parts, am_parts = [], []
    hpg = None
    for j in range(NP // CHUNK):
        sl = slice(j * CHUNK, (j + 1) * CHUNK)
        ax1 = aT[0:1, sl]
        ay1 = aT[1:2, sl]
        ax2 = aT[2:3, sl]
        ay2 = aT[3:4, sl]
        iw = jnp.maximum(jnp.minimum(gx2, ax2) - jnp.maximum(gx1, ax1), 0.0)
        ih = jnp.maximum(jnp.minimum(gy2, ay2) - jnp.maximum(gy1, ay1), 0.0)
        inter = iw * ih
        area_a = (ax2 - ax1) * (ay2 - ay1)
        iou = inter / (area_g + area_a - inter)
        lane = jax.lax.broadcasted_iota(jnp.int32, (1, CHUNK), 1) + j * CHUNK
        iou = jnp.where(lane < N, iou, -1.0)
        iou_ref[:, pl.ds(j * CHUNK, CHUNK)] = iou
        mv_c = jnp.max(iou, axis=0, keepdims=True)
        gi_c = jax.lax.broadcasted_iota(jnp.int32, (G, CHUNK), 0)
        am_parts.append(jnp.min(jnp.where(iou == mv_c, gi_c, G), axis=0,
                                keepdims=True))
        mv_parts.append(mv_c)
        hpg_c = jnp.max(iou, axis=1, keepdims=True)
        hpg = hpg_c if hpg is None else jnp.maximum(hpg, hpg_c)

    mv = jnp.concatenate(mv_parts, axis=1)
    am = jnp.concatenate(am_parts, axis=1)
    lq = jnp.max(iou_ref[:] - hpg, axis=0, keepdims=True) == 0.0

    matches = jnp.where(mv < BG, -1, jnp.where(mv < FG, -2, am))
    mi = jnp.where(lq, am, matches)
    quality = jnp.where(mi >= 0, mv, -jnp.inf)

    q2 = jnp.reshape(quality, (R, C))
    key = jax.lax.bitcast_convert_type(q2, jnp.int32)

    def sbody(_, lohi):
        lo, hi = lohi
        step = jnp.maximum((hi - lo) // 8, 1)
        new_lo, new_hi = lo, hi
        for k in range(1, 8):
            mk = lo + k * step
            ck = jnp.sum(jnp.where(key >= mk, 1, 0))
            ge = ck >= TOP_K_N
            new_lo = jnp.where(ge, mk, new_lo)
            new_hi = jnp.where(ge, new_hi, jnp.minimum(new_hi, mk))
        return new_lo, new_hi

    t, _ = jax.lax.fori_loop(
        0, 11, sbody, (jnp.int32(-1), jnp.int32(0x3F800001)))

    S = key > t
    T = key == t

    def excl_prefix(m):
        x0 = m.astype(jnp.int32)
        x = x0
        s = 1
        while s < C:
            x = x + jnp.concatenate(
                [jnp.zeros((R, s), jnp.int32), x[:, :C - s]], axis=1)
            s *= 2
        rt = x[:, C - 1:C]
        o = rt
        s = 1
        while s < R:
            o = o + jnp.concatenate(
                [jnp.zeros((s, 1), jnp.int32), o[:R - s, :]], axis=0)
            s *= 2
        return x + (o - rt) - x0

    sizeS = jnp.sum(S.astype(jnp.int32))
    tie_rank = sizeS + excl_prefix(T)
    U = S | (T & (tie_rank < TOP_K_N))
    P_sel = jnp.sum(U.astype(jnp.int32))
    slot = jnp.where(S, excl_prefix(S), tie_rank)
    slot_row = jnp.reshape(jnp.where(U, slot, -1), (1, NP))

    u_row = slot_row >= 0
    labels_row = jnp.where(u_row, 1.0, jnp.where(mi == -1, 0.0, -1.0))
    labels_ref[...] = jnp.reshape(labels_row[:, :N], (N,))

    gi = jax.lax.broadcasted_iota(jnp.int32, (G, NP), 0)
    onehot_ref[0:G, :] = jnp.where(gi == slot_row, 1.0, 0.0
                                   ).astype(jnp.bfloat16)
    onehot_ref[G:CS, :] = jnp.where(gi[:CS - G, :] == slot_row - G, 1.0, 0.0
                                    ).astype(jnp.bfloat16)

    key_row = jnp.reshape(key, (1, NP))
    ni_row = jax.lax.broadcasted_iota(jnp.int32, (1, NP), 1)

    def byte_row(x):
        return (x & 255).astype(jnp.float32).astype(jnp.bfloat16)

    x8 = jnp.concatenate([
        byte_row(key_row), byte_row(key_row >> 8),
        byte_row(key_row >> 16), byte_row(key_row >> 24),
        byte_row(ni_row), byte_row(ni_row >> 8),
        byte_row(am), jnp.zeros((1, NP), jnp.bfloat16),
    ], axis=0)
    cand = jax.lax.dot_general(
        onehot_ref[:], x8, (((1,), (1,)), ((), ())),
        preferred_element_type=jnp.float32)

    def col(jc):
        return cand[:, jc:jc + 1].astype(jnp.int32)

    klo_c = col(0) | (col(1) << 8)
    khi_c = col(2) | (col(3) << 8)
    i_c = col(4) | (col(5) << 8)
    g_c = col(6)

    khiT = jnp.reshape(khi_c, (1, CS))
    kloT = jnp.reshape(klo_c, (1, CS))
    iT = jnp.reshape(i_c, (1, CS))
    real_r = jax.lax.broadcasted_iota(jnp.int32, (CS, 1), 0) < P_sel
    gtm = ((khi_c > khiT)
           | ((khi_c == khiT)
              & ((klo_c > kloT)
                 | ((klo_c == kloT) & (i_c < iT)))))
    rank_flat = jnp.sum((gtm & real_r).astype(jnp.int32), axis=0,
                        keepdims=True)
    real_l = jax.lax.broadcasted_iota(jnp.int32, (1, CS), 1) < P_sel
    rank_c_row = jnp.where(real_l, rank_flat, BIG)

    kiota = jax.lax.broadcasted_iota(jnp.int32, (TOP_K_N, CS), 0)
    onehot_kc = jnp.where(kiota == rank_c_row, 1.0, 0.0).astype(jnp.bfloat16)
    giota_l = jax.lax.broadcasted_iota(jnp.int32, (CS, G), 1)
    onehot_cg = jnp.where(giota_l == g_c, 1.0, 0.0).astype(jnp.bfloat16)
    sel_g = jax.lax.dot_general(
        onehot_kc, onehot_cg, (((1,), (0,)), ((), ())),
        preferred_element_type=jnp.float32)
    boxes_ref[...] = jax.lax.dot_general(
        sel_g, gt_ref[:], (((1,), (0,)), ((), ())),
        precision=jax.lax.Precision.HIGHEST,
        preferred_element_type=jnp.float32)


def kernel(anchors, gt_boxes):
    labels, boxes = pl.pallas_call(
        _matcher_body,
        out_shape=(
            jax.ShapeDtypeStruct((N,), jnp.float32),
            jax.ShapeDtypeStruct((TOP_K_N, 4), jnp.float32),
        ),
        scratch_shapes=[pltpu.VMEM((G, NP), jnp.float32),
                        pltpu.VMEM((CS, NP), jnp.bfloat16)],
    )(anchors, gt_boxes)
    return labels, boxes


# speedup vs baseline: 1.4115x; 1.4115x over previous
import jax
import jax.numpy as jnp
from jax.experimental import pallas as pl
from jax.experimental.pallas import tpu as pltpu

N = 20000
NP = 20480
G = 128
TOP_K_N = 200
R, C = 8, 2560
CHUNK = 5120
CS = 208
BIG = 2 ** 30
FG = 0.8
BG = 0.1


def _matcher_body(aT_ref, gt_ref, labels_ref, boxes_ref, iou_ref, onehot_ref):
    gx1 = gt_ref[:, 0:1]
    gy1 = gt_ref[:, 1:2]
    gx2 = gt_ref[:, 2:3]
    gy2 = gt_ref[:, 3:4]
    area_g = (gx2 - gx1) * (gy2 - gy1)

    mv_parts, am_parts = [], []
    hpg = None
    for j in range(NP // CHUNK):
        sl = pl.ds(j * CHUNK, CHUNK)
        ax1 = aT_ref[0:1, sl]
        ay1 = aT_ref[1:2, sl]
        ax2 = aT_ref[2:3, sl]
        ay2 = aT_ref[3:4, sl]
        iw = jnp.maximum(jnp.minimum(gx2, ax2) - jnp.maximum(gx1, ax1), 0.0)
        ih = jnp.maximum(jnp.minimum(gy2, ay2) - jnp.maximum(gy1, ay1), 0.0)
        inter = iw * ih
        area_a = (ax2 - ax1) * (ay2 - ay1)
        iou = inter / (area_g + area_a - inter)
        lane = jax.lax.broadcasted_iota(jnp.int32, (1, CHUNK), 1) + j * CHUNK
        iou = jnp.where(lane < N, iou, -1.0)
        iou_ref[:, sl] = iou
        mv_c = jnp.max(iou, axis=0, keepdims=True)
        gi_c = jax.lax.broadcasted_iota(jnp.int32, (G, CHUNK), 0)
        am_parts.append(jnp.min(jnp.where(iou == mv_c, gi_c, G), axis=0,
                                keepdims=True))
        mv_parts.append(mv_c)
        hpg_c = jnp.max(iou, axis=1, keepdims=True)
        hpg = hpg_c if hpg is None else jnp.maximum(hpg, hpg_c)

    mv = jnp.concatenate(mv_parts, axis=1)
    am = jnp.concatenate(am_parts, axis=1)
    lq = jnp.max(iou_ref[:] - hpg, axis=0, keepdims=True) == 0.0

    matches = jnp.where(mv < BG, -1, jnp.where(mv < FG, -2, am))
    mi = jnp.where(lq, am, matches)
    quality = jnp.where(mi >= 0, mv, -jnp.inf)

    q2 = jnp.reshape(quality, (R, C))
    key = jax.lax.bitcast_convert_type(q2, jnp.int32)

    def sbody(_, lohi):
        lo, hi = lohi
        step = jnp.maximum((hi - lo) // 8, 1)
        new_lo, new_hi = lo, hi
        for k in range(1, 8):
            mk = lo + k * step
            ck = jnp.sum(jnp.where(key >= mk, 1, 0))
            ge = ck >= TOP_K_N
            new_lo = jnp.where(ge, mk, new_lo)
            new_hi = jnp.where(ge, new_hi, jnp.minimum(new_hi, mk))
        return new_lo, new_hi

    t, _ = jax.lax.fori_loop(
        0, 11, sbody, (jnp.int32(-1), jnp.int32(0x3F800001)))

    S = key > t
    T = key == t

    def excl_prefix(m):
        x0 = m.astype(jnp.int32)
        x = x0
        s = 1
        while s < C:
            x = x + jnp.concatenate(
                [jnp.zeros((R, s), jnp.int32), x[:, :C - s]], axis=1)
            s *= 2
        rt = x[:, C - 1:C]
        o = rt
        s = 1
        while s < R:
            o = o + jnp.concatenate(
                [jnp.zeros((s, 1), jnp.int32), o[:R - s, :]], axis=0)
            s *= 2
        return x + (o - rt) - x0

    sizeS = jnp.sum(S.astype(jnp.int32))
    tie_rank = sizeS + excl_prefix(T)
    U = S | (T & (tie_rank < TOP_K_N))
    P_sel = jnp.sum(U.astype(jnp.int32))
    slot = jnp.where(S, excl_prefix(S), tie_rank)
    slot_row = jnp.reshape(jnp.where(U, slot, -1), (1, NP))

    u_row = slot_row >= 0
    labels_row = jnp.where(u_row, 1.0, jnp.where(mi == -1, 0.0, -1.0))
    labels_ref[...] = jnp.reshape(labels_row[:, :N], (N,))

    gi = jax.lax.broadcasted_iota(jnp.int32, (G, NP), 0)
    onehot_ref[0:G, :] = jnp.where(gi == slot_row, 1.0, 0.0
                                   ).astype(jnp.bfloat16)
    onehot_ref[G:CS, :] = jnp.where(gi[:CS - G, :] == slot_row - G, 1.0, 0.0
                                    ).astype(jnp.bfloat16)

    key_row = jnp.reshape(key, (1, NP))
    ni_row = jax.lax.broadcasted_iota(jnp.int32, (1, NP), 1)

    def byte_row(x):
        return (x & 255).astype(jnp.float32).astype(jnp.bfloat16)

    x8 = jnp.concatenate([
        byte_row(key_row), byte_row(key_row >> 8),
        byte_row(key_row >> 16), byte_row(key_row >> 24),
        byte_row(ni_row), byte_row(ni_row >> 8),
        byte_row(am), jnp.zeros((1, NP), jnp.bfloat16),
    ], axis=0)
    cand = jax.lax.dot_general(
        onehot_ref[:], x8, (((1,), (1,)), ((), ())),
        preferred_element_type=jnp.float32)

    def col(jc):
        return cand[:, jc:jc + 1].astype(jnp.int32)

    klo_c = col(0) | (col(1) << 8)
    khi_c = col(2) | (col(3) << 8)
    i_c = col(4) | (col(5) << 8)
    g_c = col(6)

    khiT = jnp.reshape(khi_c, (1, CS))
    kloT = jnp.reshape(klo_c, (1, CS))
    iT = jnp.reshape(i_c, (1, CS))
    real_r = jax.lax.broadcasted_iota(jnp.int32, (CS, 1), 0) < P_sel
    gtm = ((khi_c > khiT)
           | ((khi_c == khiT)
              & ((klo_c > kloT)
                 | ((klo_c == kloT) & (i_c < iT)))))
    rank_flat = jnp.sum((gtm & real_r).astype(jnp.int32), axis=0,
                        keepdims=True)
    real_l = jax.lax.broadcasted_iota(jnp.int32, (1, CS), 1) < P_sel
    rank_c_row = jnp.where(real_l, rank_flat, BIG)

    kiota = jax.lax.broadcasted_iota(jnp.int32, (TOP_K_N, CS), 0)
    onehot_kc = jnp.where(kiota == rank_c_row, 1.0, 0.0).astype(jnp.bfloat16)
    giota_l = jax.lax.broadcasted_iota(jnp.int32, (CS, G), 1)
    onehot_cg = jnp.where(giota_l == g_c, 1.0, 0.0).astype(jnp.bfloat16)
    sel_g = jax.lax.dot_general(
        onehot_kc, onehot_cg, (((1,), (0,)), ((), ())),
        preferred_element_type=jnp.float32)
    boxes_ref[...] = jax.lax.dot_general(
        sel_g, gt_ref[:], (((1,), (0,)), ((), ())),
        precision=jax.lax.Precision.HIGHEST,
        preferred_element_type=jnp.float32)


def kernel(anchors, gt_boxes):
    aT = jnp.pad(jnp.transpose(anchors), ((0, 0), (0, NP - N)))
    labels, boxes = pl.pallas_call(
        _matcher_body,
        out_shape=(
            jax.ShapeDtypeStruct((N,), jnp.float32),
            jax.ShapeDtypeStruct((TOP_K_N, 4), jnp.float32),
        ),
        scratch_shapes=[pltpu.VMEM((G, NP), jnp.float32),
                        pltpu.VMEM((CS, NP), jnp.bfloat16)],
    )(aT, gt_boxes)
    return labels, boxes
